# baseline (speedup 1.0000x reference)
import functools

import jax
import jax.numpy as jnp
from jax import lax
from jax.experimental import pallas as pl
from jax.experimental.pallas import tpu as pltpu
from jax.experimental.pallas import tpu_sc as plsc

T, N, E, D, G, H = 6, 10000, 320000, 128, 128, 128
NPAD = 10240
NC, NS = 2, 16
TPC = T // NC
EPT = E // NS
KE = 160
NCHUNK = EPT // KE
RPT = NPAD // NS
ZR = 40

_PREC = jax.lax.Precision.DEFAULT


def _mesh():
    return plsc.VectorSubcoreMesh(core_axis_name="c", subcore_axis_name="s")


KD = 2000
NCHUNK_D = EPT // KD


def _sc_deg_body(dst_hbm, deg_hbm, ones_v, idx0, idx1, zrow_v, l0, l1, acc_s):
    c = lax.axis_index("c")
    s = lax.axis_index("s")

    def _fill_ones(i, _):
        ones_v[pl.ds(i * 16, 16)] = jnp.full((16,), 1.0, jnp.float32)
        return ()

    def _fill_zeros(i, _):
        zrow_v[pl.ds(i * 16, 16)] = jnp.zeros((16,), jnp.float32)
        return ()

    lax.fori_loop(0, KD // 16, _fill_ones, ())
    lax.fori_loop(0, RPT // 16, _fill_zeros, ())

    for ti in range(TPC):
        pltpu.sync_copy(zrow_v, acc_s.at[pl.ds(ti * NPAD + s * RPT, RPT)])
    plsc.subcore_barrier()

    clamp = T * E - KD

    for ti in range(TPC):
        t = c * TPC + ti
        ebase = t * E + s * EPT
        acc_t = acc_s.at[pl.ds(ti * NPAD, NPAD)]

        def _start(ck, ib, sem):
            b = pl.multiple_of(jnp.minimum(ebase + ck * KD, clamp), 8)
            pltpu.async_copy(dst_hbm.at[pl.ds(b, KD)], ib, sem)

        def _wait(ib, sem):
            pltpu.make_async_copy(dst_hbm.at[pl.ds(0, KD)], ib, sem).wait()

        _start(0, idx0, l0)
        _start(1, idx1, l1)

        def _pair(k, _):
            a = 2 * k
            _wait(idx0, l0)
            pltpu.sync_copy(ones_v, acc_t.at[idx0], add=True)
            _start(a + 2, idx0, l0)
            _wait(idx1, l1)
            pltpu.sync_copy(ones_v, acc_t.at[idx1], add=True)
            _start(a + 3, idx1, l1)
            return ()

        lax.fori_loop(0, NCHUNK_D // 2, _pair, ())
        _wait(idx0, l0)
        _wait(idx1, l1)
    plsc.subcore_barrier()

    for ti in range(TPC):
        t = c * TPC + ti
        dst_off = pl.multiple_of(t * NPAD + s * RPT, 8)
        pltpu.sync_copy(acc_s.at[pl.ds(ti * NPAD + s * RPT, RPT)],
                        deg_hbm.at[pl.ds(dst_off, RPT)])


def _deg_call(dst_flat):
    return pl.kernel(
        _sc_deg_body,
        out_type=jax.ShapeDtypeStruct((T * NPAD,), jnp.float32),
        mesh=_mesh(),
        scratch_types=[
            pltpu.VMEM((KD,), jnp.float32),
            pltpu.VMEM((KD,), jnp.int32),
            pltpu.VMEM((KD,), jnp.int32),
            pltpu.VMEM((RPT,), jnp.float32),
            pltpu.SemaphoreType.DMA,
            pltpu.SemaphoreType.DMA,
            pltpu.VMEM_SHARED((TPC * NPAD,), jnp.float32),
        ],
    )(dst_flat)


def _sc_agg_body(tbase, src_hbm, dst_hbm, tab_hbm, out_hbm,
                 idxs0, idxd0, idxs1, idxd1, rows0, rows1, zbuf_v,
                 ls0, ld0, ls1, ld1, g0, g1, acc_s):
    c = lax.axis_index("c")
    s = lax.axis_index("s")

    def _fill(i, _):
        j = i // (128 // 16)
        k = (i % (128 // 16)) * 16
        zbuf_v[j, pl.ds(k, 16)] = jnp.zeros((16,), jnp.float32)
        return ()

    lax.fori_loop(0, ZR * (128 // 16), _fill, ())

    clamp = T * E - KE

    for j in range(RPT // ZR):
        zoff = pl.multiple_of(s * RPT + j * ZR, 8)
        pltpu.sync_copy(zbuf_v, acc_s.at[pl.ds(zoff, ZR), :])
    plsc.subcore_barrier()

    if True:
        t = tbase + c
        row0 = pl.multiple_of(c * NPAD, 8)
        tab_t = tab_hbm.at[pl.ds(row0, NPAD), :]
        ebase = t * E + s * EPT

        def _start_load(ck, ibs, ibd, sls, sld):
            b = pl.multiple_of(jnp.minimum(ebase + ck * KE, clamp), 8)
            pltpu.async_copy(src_hbm.at[pl.ds(b, KE)], ibs, sls)
            pltpu.async_copy(dst_hbm.at[pl.ds(b, KE)], ibd, sld)

        def _wait_load(ibs, ibd, sls, sld):
            pltpu.make_async_copy(src_hbm.at[pl.ds(0, KE)], ibs, sls).wait()
            pltpu.make_async_copy(dst_hbm.at[pl.ds(0, KE)], ibd, sld).wait()

        def _start_gather(ibs, rows, g):
            pltpu.async_copy(tab_t.at[ibs], rows, g)

        def _wait_gather(rows, g):
            pltpu.make_async_copy(tab_hbm.at[pl.ds(0, KE), :], rows, g).wait()

        def _scatter(rows, ibd):
            pltpu.sync_copy(rows, acc_s.at[ibd], add=True)

        _start_load(0, idxs0, idxd0, ls0, ld0)
        _start_load(1, idxs1, idxd1, ls1, ld1)
        _wait_load(idxs0, idxd0, ls0, ld0)
        _start_gather(idxs0, rows0, g0)

        def _pair(k, _):
            a = 2 * k
            _wait_load(idxs1, idxd1, ls1, ld1)
            _wait_gather(rows0, g0)
            _start_gather(idxs1, rows1, g1)
            _scatter(rows0, idxd0)
            _start_load(a + 2, idxs0, idxd0, ls0, ld0)
            _wait_load(idxs0, idxd0, ls0, ld0)
            _wait_gather(rows1, g1)
            _start_gather(idxs0, rows0, g0)
            _scatter(rows1, idxd1)
            _start_load(a + 3, idxs1, idxd1, ls1, ld1)
            return ()

        lax.fori_loop(0, NCHUNK // 2, _pair, ())
        _wait_load(idxs1, idxd1, ls1, ld1)
        _wait_gather(rows0, g0)
        _scatter(rows0, idxd0)
        plsc.subcore_barrier()

        out_off = pl.multiple_of(c * NPAD + s * RPT, 8)
        acc_off = pl.multiple_of(s * RPT, 8)
        pltpu.sync_copy(acc_s.at[pl.ds(acc_off, RPT), :],
                        out_hbm.at[pl.ds(out_off, RPT), :])


def _agg_call(src_flat, dst_flat, tab_pair, tbase):
    return pl.kernel(
        functools.partial(_sc_agg_body, tbase),
        out_type=jax.ShapeDtypeStruct((2 * NPAD, 128), jnp.float32),
        mesh=_mesh(),
        scratch_types=[
            pltpu.VMEM((KE,), jnp.int32),
            pltpu.VMEM((KE,), jnp.int32),
            pltpu.VMEM((KE,), jnp.int32),
            pltpu.VMEM((KE,), jnp.int32),
            pltpu.VMEM((KE, 128), jnp.float32),
            pltpu.VMEM((KE, 128), jnp.float32),
            pltpu.VMEM((ZR, 128), jnp.float32),
            pltpu.SemaphoreType.DMA,
            pltpu.SemaphoreType.DMA,
            pltpu.SemaphoreType.DMA,
            pltpu.SemaphoreType.DMA,
            pltpu.SemaphoreType.DMA,
            pltpu.SemaphoreType.DMA,
            pltpu.VMEM_SHARED((NPAD, 128), jnp.float32),
        ],
    )(src_flat, dst_flat, tab_pair)


BN = 2048
BN2 = 1024


def _tc_pre_body(x_ref, w_ref, deg_ref, out_ref):
    dinv = lax.rsqrt(deg_ref[0] + 1.0)
    h = jnp.dot(x_ref[0], w_ref[...], preferred_element_type=jnp.float32,
                precision=_PREC)
    out_ref[0] = h * dinv


def _tc_pre(xp, w1, deg):
    grid = (T, NPAD // BN)
    return pl.pallas_call(
        _tc_pre_body,
        grid=grid,
        in_specs=[
            pl.BlockSpec((1, BN, D), lambda t, i: (t, i, 0)),
            pl.BlockSpec((D, G), lambda t, i: (0, 0)),
            pl.BlockSpec((1, BN, 1), lambda t, i: (t, i, 0)),
        ],
        out_specs=pl.BlockSpec((1, BN, G), lambda t, i: (t, i, 0)),
        out_shape=jax.ShapeDtypeStruct((T, NPAD, G), jnp.float32),
        compiler_params=pltpu.CompilerParams(
            dimension_semantics=("parallel", "parallel")),
    )(xp, w1, deg)


def _tc_mid_body(ht1_ref, acc_ref, deg_ref, w2_ref, b1_ref, h1_out, ht2_out):
    dinv = lax.rsqrt(deg_ref[0] + 1.0)
    h1 = jnp.maximum(dinv * (acc_ref[0] + ht1_ref[0]) + b1_ref[...], 0.0)
    h1_out[0] = h1
    ht2_out[0] = jnp.dot(h1, w2_ref[...], preferred_element_type=jnp.float32,
                         precision=_PREC) * dinv


def _tc_mid(ht1, acc1, deg, w2, b1):
    grid = (2, NPAD // BN)
    return pl.pallas_call(
        _tc_mid_body,
        grid=grid,
        in_specs=[
            pl.BlockSpec((1, BN, G), lambda t, i: (t, i, 0)),
            pl.BlockSpec((1, BN, G), lambda t, i: (t, i, 0)),
            pl.BlockSpec((1, BN, 1), lambda t, i: (t, i, 0)),
            pl.BlockSpec((G, G), lambda t, i: (0, 0)),
            pl.BlockSpec((1, G), lambda t, i: (0, 0)),
        ],
        out_specs=[
            pl.BlockSpec((1, BN, G), lambda t, i: (t, i, 0)),
            pl.BlockSpec((1, BN, G), lambda t, i: (t, i, 0)),
        ],
        out_shape=[
            jax.ShapeDtypeStruct((2, NPAD, G), jnp.float32),
            jax.ShapeDtypeStruct((2, NPAD, G), jnp.float32),
        ],
        compiler_params=pltpu.CompilerParams(
            dimension_semantics=("parallel", "parallel")),
    )(ht1, acc1, deg, w2, b1)


def _tc_rnn_body(h1_ref, ht2_ref, acc2_ref, degt_ref, b2_ref, wih_ref,
                 whh_ref, bih_ref, bhh_ref, wa_ref, ba_ref, wp1_ref, bp1_ref,
                 wp2_ref, bp2_ref, out_ref):
    dinvt = lax.rsqrt(degt_ref[...] + 1.0)
    wih = wih_ref[...]
    whh = whh_ref[...]
    bih = bih_ref[...]
    bhh = bhh_ref[...]
    h = jnp.zeros((BN2, H), jnp.float32)
    hs = []
    for t in range(T):
        dinv = dinvt[:, t:t + 1]
        h2 = jnp.maximum(dinv * (acc2_ref[t] + ht2_ref[t]) + b2_ref[...], 0.0)
        xt = jnp.concatenate([h1_ref[t], h2], axis=1)
        gi = lax.dot_general(xt, wih, (((1,), (1,)), ((), ())),
                             preferred_element_type=jnp.float32,
                             precision=_PREC) + bih
        gh = lax.dot_general(h, whh, (((1,), (1,)), ((), ())),
                             preferred_element_type=jnp.float32,
                             precision=_PREC) + bhh
        r = jax.nn.sigmoid(gi[:, :H] + gh[:, :H])
        z = jax.nn.sigmoid(gi[:, H:2 * H] + gh[:, H:2 * H])
        n = jnp.tanh(gi[:, 2 * H:] + r * gh[:, 2 * H:])
        h = (1.0 - z) * n + z * h
        hs.append(h)
    scores = jnp.concatenate(
        [jnp.tanh(jnp.dot(ht, wa_ref[...], preferred_element_type=jnp.float32,
                          precision=_PREC) + ba_ref[...]) for ht in hs],
        axis=1)
    wts = jax.nn.softmax(scores, axis=1)
    ctx = hs[0] * wts[:, 0:1]
    for t in range(1, T):
        ctx = ctx + hs[t] * wts[:, t:t + 1]
    hid = jnp.maximum(
        jnp.dot(ctx, wp1_ref[...], preferred_element_type=jnp.float32,
                precision=_PREC) + bp1_ref[...], 0.0)
    out_ref[...] = jnp.dot(hid, wp2_ref[...],
                           preferred_element_type=jnp.float32,
                           precision=_PREC) + bp2_ref[...]


def _tc_rnn(h1, ht2, acc2, degt, b2, wih, whh, bih, bhh, wa, ba, wp1, bp1,
            wp2, bp2):
    grid = (NPAD // BN2,)
    full = lambda shape: pl.BlockSpec(shape, lambda i: tuple(0 for _ in shape))
    return pl.pallas_call(
        _tc_rnn_body,
        grid=grid,
        in_specs=[
            pl.BlockSpec((T, BN2, G), lambda i: (0, i, 0)),
            pl.BlockSpec((T, BN2, G), lambda i: (0, i, 0)),
            pl.BlockSpec((T, BN2, G), lambda i: (0, i, 0)),
            pl.BlockSpec((BN2, T), lambda i: (i, 0)),
            full((1, G)),
            full((3 * H, 2 * G)),
            full((3 * H, H)),
            full((1, 3 * H)),
            full((1, 3 * H)),
            full((H, 1)),
            full((1, 1)),
            full((H, 16)),
            full((1, 16)),
            full((16, 1)),
            full((1, 1)),
        ],
        out_specs=pl.BlockSpec((BN2, 1), lambda i: (i, 0)),
        out_shape=jax.ShapeDtypeStruct((NPAD, 1), jnp.float32),
        compiler_params=pltpu.CompilerParams(
            dimension_semantics=("parallel",)),
    )(h1, ht2, acc2, degt, b2, wih, whh, bih, bhh, wa, ba, wp1, bp1, wp2, bp2)


def kernel(x, edge_index, W1, b1, W2, b2, Wih, Whh, bih, bhh, Wa, ba, Wp1,
           bp1, Wp2, bp2):
    src = edge_index[:, 0, :].reshape(T * E)
    dst = edge_index[:, 1, :].reshape(T * E)
    xp = jnp.pad(x, ((0, 0), (0, NPAD - N), (0, 0)))

    deg = _deg_call(dst).reshape(T, NPAD)
    deg3 = deg[..., None]
    ht1 = _tc_pre(xp, W1, deg3)

    h1_pairs, ht2_pairs, acc2_pairs = [], [], []
    for tb in (0, 2, 4):
        a1 = _agg_call(src, dst, ht1[tb:tb + 2].reshape(2 * NPAD, G), tb)
        h1p, ht2p = _tc_mid(ht1[tb:tb + 2], a1.reshape(2, NPAD, G),
                            deg3[tb:tb + 2], W2, b1.reshape(1, G))
        h1_pairs.append(h1p)
        ht2_pairs.append(ht2p)
    for tb in (0, 2, 4):
        a2 = _agg_call(src, dst,
                       ht2_pairs[tb // 2].reshape(2 * NPAD, G), tb)
        acc2_pairs.append(a2.reshape(2, NPAD, G))
    h1 = jnp.concatenate(h1_pairs, axis=0)
    ht2 = jnp.concatenate(ht2_pairs, axis=0)
    acc2 = jnp.concatenate(acc2_pairs, axis=0)
    out = _tc_rnn(h1, ht2, acc2, deg.T, b2.reshape(1, G), Wih, Whh,
                  bih.reshape(1, 3 * H), bhh.reshape(1, 3 * H), Wa,
                  ba.reshape(1, 1), Wp1, bp1.reshape(1, 16), Wp2,
                  bp2.reshape(1, 1))
    return out[:N]


# speedup vs baseline: 1.0355x; 1.0355x over previous
import functools

import jax
import jax.numpy as jnp
from jax import lax
from jax.experimental import pallas as pl
from jax.experimental.pallas import tpu as pltpu
from jax.experimental.pallas import tpu_sc as plsc

T, N, E, D, G, H = 6, 10000, 320000, 128, 128, 128
NPAD = 10240
NC, NS = 2, 16
TPC = T // NC
EPT = E // NS
KE = 160
NCHUNK = EPT // KE
RPT = NPAD // NS
ZR = 40

_PREC = jax.lax.Precision.DEFAULT


def _mesh():
    return plsc.VectorSubcoreMesh(core_axis_name="c", subcore_axis_name="s")


KD = 2000
NCHUNK_D = EPT // KD


def _sc_deg_body(dst_hbm, deg_hbm, ones_v, idx0, idx1, zrow_v, l0, l1, acc_s):
    c = lax.axis_index("c")
    s = lax.axis_index("s")

    def _fill_ones(i, _):
        ones_v[pl.ds(i * 16, 16)] = jnp.full((16,), 1.0, jnp.float32)
        return ()

    def _fill_zeros(i, _):
        zrow_v[pl.ds(i * 16, 16)] = jnp.zeros((16,), jnp.float32)
        return ()

    lax.fori_loop(0, KD // 16, _fill_ones, ())
    lax.fori_loop(0, RPT // 16, _fill_zeros, ())

    for ti in range(TPC):
        pltpu.sync_copy(zrow_v, acc_s.at[pl.ds(ti * NPAD + s * RPT, RPT)])
    plsc.subcore_barrier()

    clamp = T * E - KD

    for ti in range(TPC):
        t = c * TPC + ti
        ebase = t * E + s * EPT
        acc_t = acc_s.at[pl.ds(ti * NPAD, NPAD)]

        def _start(ck, ib, sem):
            b = pl.multiple_of(jnp.minimum(ebase + ck * KD, clamp), 8)
            pltpu.async_copy(dst_hbm.at[pl.ds(b, KD)], ib, sem)

        def _wait(ib, sem):
            pltpu.make_async_copy(dst_hbm.at[pl.ds(0, KD)], ib, sem).wait()

        _start(0, idx0, l0)
        _start(1, idx1, l1)

        def _pair(k, _):
            a = 2 * k
            _wait(idx0, l0)
            pltpu.sync_copy(ones_v, acc_t.at[idx0], add=True)
            _start(a + 2, idx0, l0)
            _wait(idx1, l1)
            pltpu.sync_copy(ones_v, acc_t.at[idx1], add=True)
            _start(a + 3, idx1, l1)
            return ()

        lax.fori_loop(0, NCHUNK_D // 2, _pair, ())
        _wait(idx0, l0)
        _wait(idx1, l1)
    plsc.subcore_barrier()

    for ti in range(TPC):
        t = c * TPC + ti
        dst_off = pl.multiple_of(t * NPAD + s * RPT, 8)
        pltpu.sync_copy(acc_s.at[pl.ds(ti * NPAD + s * RPT, RPT)],
                        deg_hbm.at[pl.ds(dst_off, RPT)])


def _deg_call(dst_flat):
    return pl.kernel(
        _sc_deg_body,
        out_type=jax.ShapeDtypeStruct((T * NPAD,), jnp.float32),
        mesh=_mesh(),
        scratch_types=[
            pltpu.VMEM((KD,), jnp.float32),
            pltpu.VMEM((KD,), jnp.int32),
            pltpu.VMEM((KD,), jnp.int32),
            pltpu.VMEM((RPT,), jnp.float32),
            pltpu.SemaphoreType.DMA,
            pltpu.SemaphoreType.DMA,
            pltpu.VMEM_SHARED((TPC * NPAD,), jnp.float32),
        ],
    )(dst_flat)


def _sc_agg_body(src_hbm, dst_hbm, tab_hbm, out_hbm,
                 idxs0, idxd0, idxs1, idxd1, rows0, rows1, zbuf_v,
                 ls0, ld0, ls1, ld1, g0, g1, acc_s):
    c = lax.axis_index("c")
    s = lax.axis_index("s")

    def _fill(i, _):
        j = i // (128 // 16)
        k = (i % (128 // 16)) * 16
        zbuf_v[j, pl.ds(k, 16)] = jnp.zeros((16,), jnp.float32)
        return ()

    lax.fori_loop(0, ZR * (128 // 16), _fill, ())

    clamp = T * E - KE

    for j in range(RPT // ZR):
        zoff = pl.multiple_of(s * RPT + j * ZR, 8)
        pltpu.sync_copy(zbuf_v, acc_s.at[pl.ds(zoff, ZR), :])
    plsc.subcore_barrier()

    for ti in range(TPC):
        t = c * TPC + ti
        row0 = pl.multiple_of(t * NPAD, 8)
        tab_t = tab_hbm.at[pl.ds(row0, NPAD), :]
        ebase = t * E + s * EPT

        def _start_load(ck, ibs, ibd, sls, sld):
            b = pl.multiple_of(jnp.minimum(ebase + ck * KE, clamp), 8)
            pltpu.async_copy(src_hbm.at[pl.ds(b, KE)], ibs, sls)
            pltpu.async_copy(dst_hbm.at[pl.ds(b, KE)], ibd, sld)

        def _wait_load(ibs, ibd, sls, sld):
            pltpu.make_async_copy(src_hbm.at[pl.ds(0, KE)], ibs, sls).wait()
            pltpu.make_async_copy(dst_hbm.at[pl.ds(0, KE)], ibd, sld).wait()

        def _start_gather(ibs, rows, g):
            pltpu.async_copy(tab_t.at[ibs], rows, g)

        def _wait_gather(rows, g):
            pltpu.make_async_copy(tab_hbm.at[pl.ds(0, KE), :], rows, g).wait()

        def _scatter(rows, ibd):
            pltpu.sync_copy(rows, acc_s.at[ibd], add=True)

        _start_load(0, idxs0, idxd0, ls0, ld0)
        _start_load(1, idxs1, idxd1, ls1, ld1)
        _wait_load(idxs0, idxd0, ls0, ld0)
        _start_gather(idxs0, rows0, g0)

        def _pair(k, _):
            a = 2 * k
            _wait_load(idxs1, idxd1, ls1, ld1)
            _wait_gather(rows0, g0)
            _start_gather(idxs1, rows1, g1)
            _scatter(rows0, idxd0)
            _start_load(a + 2, idxs0, idxd0, ls0, ld0)
            _wait_load(idxs0, idxd0, ls0, ld0)
            _wait_gather(rows1, g1)
            _start_gather(idxs0, rows0, g0)
            _scatter(rows1, idxd1)
            _start_load(a + 3, idxs1, idxd1, ls1, ld1)
            return ()

        lax.fori_loop(0, NCHUNK // 2, _pair, ())
        _wait_load(idxs1, idxd1, ls1, ld1)
        _wait_gather(rows0, g0)
        _scatter(rows0, idxd0)
        plsc.subcore_barrier()

        out_off = pl.multiple_of(t * NPAD + s * RPT, 8)
        acc_off = pl.multiple_of(s * RPT, 8)
        pltpu.sync_copy(acc_s.at[pl.ds(acc_off, RPT), :],
                        out_hbm.at[pl.ds(out_off, RPT), :])
        if ti != TPC - 1:
            for j in range(RPT // ZR):
                zoff = pl.multiple_of(s * RPT + j * ZR, 8)
                pltpu.sync_copy(zbuf_v, acc_s.at[pl.ds(zoff, ZR), :])
        plsc.subcore_barrier()


def _agg_call(src_flat, dst_flat, tab_flat):
    return pl.kernel(
        _sc_agg_body,
        out_type=jax.ShapeDtypeStruct((T * NPAD, 128), jnp.float32),
        mesh=_mesh(),
        scratch_types=[
            pltpu.VMEM((KE,), jnp.int32),
            pltpu.VMEM((KE,), jnp.int32),
            pltpu.VMEM((KE,), jnp.int32),
            pltpu.VMEM((KE,), jnp.int32),
            pltpu.VMEM((KE, 128), jnp.float32),
            pltpu.VMEM((KE, 128), jnp.float32),
            pltpu.VMEM((ZR, 128), jnp.float32),
            pltpu.SemaphoreType.DMA,
            pltpu.SemaphoreType.DMA,
            pltpu.SemaphoreType.DMA,
            pltpu.SemaphoreType.DMA,
            pltpu.SemaphoreType.DMA,
            pltpu.SemaphoreType.DMA,
            pltpu.VMEM_SHARED((NPAD, 128), jnp.float32),
        ],
    )(src_flat, dst_flat, tab_flat)


BN = 2048
BN2 = 1024


def _tc_pre_body(x_ref, w_ref, deg_ref, out_ref):
    dinv = lax.rsqrt(deg_ref[0] + 1.0)
    h = jnp.dot(x_ref[0], w_ref[...], preferred_element_type=jnp.float32,
                precision=_PREC)
    out_ref[0] = h * dinv


def _tc_pre(xp, w1, deg):
    grid = (T, NPAD // BN)
    return pl.pallas_call(
        _tc_pre_body,
        grid=grid,
        in_specs=[
            pl.BlockSpec((1, BN, D), lambda t, i: (t, i, 0)),
            pl.BlockSpec((D, G), lambda t, i: (0, 0)),
            pl.BlockSpec((1, BN, 1), lambda t, i: (t, i, 0)),
        ],
        out_specs=pl.BlockSpec((1, BN, G), lambda t, i: (t, i, 0)),
        out_shape=jax.ShapeDtypeStruct((T, NPAD, G), jnp.float32),
        compiler_params=pltpu.CompilerParams(
            dimension_semantics=("parallel", "parallel")),
    )(xp, w1, deg)


def _tc_mid_body(ht1_ref, acc_ref, deg_ref, w2_ref, b1_ref, h1_out, ht2_out):
    dinv = lax.rsqrt(deg_ref[0] + 1.0)
    h1 = jnp.maximum(dinv * (acc_ref[0] + ht1_ref[0]) + b1_ref[...], 0.0)
    h1_out[0] = h1
    ht2_out[0] = jnp.dot(h1, w2_ref[...], preferred_element_type=jnp.float32,
                         precision=_PREC) * dinv


def _tc_mid(ht1, acc1, deg, w2, b1):
    grid = (T, NPAD // BN)
    return pl.pallas_call(
        _tc_mid_body,
        grid=grid,
        in_specs=[
            pl.BlockSpec((1, BN, G), lambda t, i: (t, i, 0)),
            pl.BlockSpec((1, BN, G), lambda t, i: (t, i, 0)),
            pl.BlockSpec((1, BN, 1), lambda t, i: (t, i, 0)),
            pl.BlockSpec((G, G), lambda t, i: (0, 0)),
            pl.BlockSpec((1, G), lambda t, i: (0, 0)),
        ],
        out_specs=[
            pl.BlockSpec((1, BN, G), lambda t, i: (t, i, 0)),
            pl.BlockSpec((1, BN, G), lambda t, i: (t, i, 0)),
        ],
        out_shape=[
            jax.ShapeDtypeStruct((T, NPAD, G), jnp.float32),
            jax.ShapeDtypeStruct((T, NPAD, G), jnp.float32),
        ],
        compiler_params=pltpu.CompilerParams(
            dimension_semantics=("parallel", "parallel")),
    )(ht1, acc1, deg, w2, b1)


def _tc_rnn_body(h1_ref, ht2_ref, acc2_ref, degt_ref, b2_ref, wih_ref,
                 whh_ref, bih_ref, bhh_ref, wa_ref, ba_ref, wp1_ref, bp1_ref,
                 wp2_ref, bp2_ref, out_ref):
    dinvt = lax.rsqrt(degt_ref[...] + 1.0)
    wih = wih_ref[...]
    whh = whh_ref[...]
    bih = bih_ref[...]
    bhh = bhh_ref[...]
    h = jnp.zeros((BN2, H), jnp.float32)
    hs = []
    for t in range(T):
        dinv = dinvt[:, t:t + 1]
        h2 = jnp.maximum(dinv * (acc2_ref[t] + ht2_ref[t]) + b2_ref[...], 0.0)
        xt = jnp.concatenate([h1_ref[t], h2], axis=1)
        gi = lax.dot_general(xt, wih, (((1,), (1,)), ((), ())),
                             preferred_element_type=jnp.float32,
                             precision=_PREC) + bih
        gh = lax.dot_general(h, whh, (((1,), (1,)), ((), ())),
                             preferred_element_type=jnp.float32,
                             precision=_PREC) + bhh
        r = jax.nn.sigmoid(gi[:, :H] + gh[:, :H])
        z = jax.nn.sigmoid(gi[:, H:2 * H] + gh[:, H:2 * H])
        n = jnp.tanh(gi[:, 2 * H:] + r * gh[:, 2 * H:])
        h = (1.0 - z) * n + z * h
        hs.append(h)
    scores = jnp.concatenate(
        [jnp.tanh(jnp.dot(ht, wa_ref[...], preferred_element_type=jnp.float32,
                          precision=_PREC) + ba_ref[...]) for ht in hs],
        axis=1)
    wts = jax.nn.softmax(scores, axis=1)
    ctx = hs[0] * wts[:, 0:1]
    for t in range(1, T):
        ctx = ctx + hs[t] * wts[:, t:t + 1]
    hid = jnp.maximum(
        jnp.dot(ctx, wp1_ref[...], preferred_element_type=jnp.float32,
                precision=_PREC) + bp1_ref[...], 0.0)
    out_ref[...] = jnp.dot(hid, wp2_ref[...],
                           preferred_element_type=jnp.float32,
                           precision=_PREC) + bp2_ref[...]


def _tc_rnn(h1, ht2, acc2, degt, b2, wih, whh, bih, bhh, wa, ba, wp1, bp1,
            wp2, bp2):
    grid = (NPAD // BN2,)
    full = lambda shape: pl.BlockSpec(shape, lambda i: tuple(0 for _ in shape))
    return pl.pallas_call(
        _tc_rnn_body,
        grid=grid,
        in_specs=[
            pl.BlockSpec((T, BN2, G), lambda i: (0, i, 0)),
            pl.BlockSpec((T, BN2, G), lambda i: (0, i, 0)),
            pl.BlockSpec((T, BN2, G), lambda i: (0, i, 0)),
            pl.BlockSpec((BN2, T), lambda i: (i, 0)),
            full((1, G)),
            full((3 * H, 2 * G)),
            full((3 * H, H)),
            full((1, 3 * H)),
            full((1, 3 * H)),
            full((H, 1)),
            full((1, 1)),
            full((H, 16)),
            full((1, 16)),
            full((16, 1)),
            full((1, 1)),
        ],
        out_specs=pl.BlockSpec((BN2, 1), lambda i: (i, 0)),
        out_shape=jax.ShapeDtypeStruct((NPAD, 1), jnp.float32),
        compiler_params=pltpu.CompilerParams(
            dimension_semantics=("parallel",)),
    )(h1, ht2, acc2, degt, b2, wih, whh, bih, bhh, wa, ba, wp1, bp1, wp2, bp2)


def kernel(x, edge_index, W1, b1, W2, b2, Wih, Whh, bih, bhh, Wa, ba, Wp1,
           bp1, Wp2, bp2):
    src = edge_index[:, 0, :].reshape(T * E)
    dst = edge_index[:, 1, :].reshape(T * E)
    xp = jnp.pad(x, ((0, 0), (0, NPAD - N), (0, 0)))

    deg = _deg_call(dst).reshape(T, NPAD)
    deg3 = deg[..., None]
    ht1 = _tc_pre(xp, W1, deg3)
    acc1 = _agg_call(src, dst, ht1.reshape(T * NPAD, G)).reshape(T, NPAD, G)
    h1, ht2 = _tc_mid(ht1, acc1, deg3, W2, b1.reshape(1, G))
    acc2 = _agg_call(src, dst, ht2.reshape(T * NPAD, G)).reshape(T, NPAD, G)
    out = _tc_rnn(h1, ht2, acc2, deg.T, b2.reshape(1, G), Wih, Whh,
                  bih.reshape(1, 3 * H), bhh.reshape(1, 3 * H), Wa,
                  ba.reshape(1, 1), Wp1, bp1.reshape(1, 16), Wp2,
                  bp2.reshape(1, 1))
    return out[:N]
